# baseline (speedup 1.0000x reference)
import functools

import jax
import jax.numpy as jnp
from jax import lax
from jax.experimental import pallas as pl
from jax.experimental.pallas import tpu as pltpu
from jax.experimental.pallas import tpu_sc as plsc

_ALPHA = 0.2
_NEG = -9e15
_LP = 64


def _make_sc_gather(n_rows, dim):
    info = plsc.get_sparse_core_info()
    nc, ns = info.num_cores, info.num_subcores
    nw = nc * ns
    assert n_rows % nw == 0
    b_per_w = n_rows // nw
    ch = 256
    assert b_per_w % ch == 0
    n_chunks = b_per_w // ch
    mesh = plsc.VectorSubcoreMesh(core_axis_name="c", subcore_axis_name="s")

    @functools.partial(
        pl.kernel,
        mesh=mesh,
        out_type=jax.ShapeDtypeStruct((n_rows, dim), jnp.float32),
        scratch_types=[
            pltpu.VMEM((b_per_w,), jnp.int32),
            pltpu.VMEM((2, ch, dim), jnp.float32),
            pltpu.SemaphoreType.DMA,
            pltpu.SemaphoreType.DMA,
        ],
    )
    def gather_kernel(table_hbm, idx_hbm, out_hbm, idx_v, rows_v, sem0, sem1):
        wid = lax.axis_index("s") * nc + lax.axis_index("c")
        base = wid * b_per_w
        sems = [sem0, sem1]
        pltpu.sync_copy(idx_hbm.at[pl.ds(base, b_per_w)], idx_v)
        copies = [None, None]
        copies[0] = pltpu.async_copy(
            table_hbm.at[idx_v.at[pl.ds(0, ch)]], rows_v.at[0], sems[0]
        )
        for c in range(n_chunks):
            nxt = c + 1
            if nxt < n_chunks:
                copies[nxt % 2] = pltpu.async_copy(
                    table_hbm.at[idx_v.at[pl.ds(nxt * ch, ch)]],
                    rows_v.at[nxt % 2],
                    sems[nxt % 2],
                )
            copies[c % 2].wait()
            pltpu.sync_copy(rows_v.at[c % 2], out_hbm.at[pl.ds(base + c * ch, ch)])

    return gather_kernel


def _make_attn_body(bb, l, d):
    def body(h_ref, adj_ref, a_ref, out_ref):
        a_rep = a_ref[...]
        adj_p = jnp.pad(adj_ref[...], ((0, 0), (0, _LP - l), (0, _LP - l)))
        col_ok = lax.broadcasted_iota(jnp.int32, (_LP, _LP), 1) < l
        for s in range(bb):
            h = h_ref[s]
            ha = jnp.concatenate([h, h, h, h], axis=0) * a_rep
            e = lax.dot_general(
                ha, h, (((1,), (1,)), ((), ())), preferred_element_type=jnp.float32
            )
            e = jnp.where(e >= 0, e, _ALPHA * e)
            adj = adj_p[s]
            att = jnp.full((_LP, _LP), _NEG, dtype=jnp.float32)
            for k in range(4):
                att = jnp.where(adj == (k + 1), e[k * _LP : (k + 1) * _LP], att)
            m = jnp.max(att, axis=-1, keepdims=True)
            p = jnp.where(col_ok, jnp.exp(att - m), 0.0)
            att = p / jnp.sum(p, axis=-1, keepdims=True)
            out = lax.dot_general(
                att, h, (((1,), (0,)), ((), ())), preferred_element_type=jnp.float32
            )
            out_ref[s] = out[:l]

    return body


def _attention_tc(h_p, adj, a_rep, bb):
    b, lp, d = h_p.shape
    l = adj.shape[-1]
    return pl.pallas_call(
        _make_attn_body(bb, l, d),
        grid=(b // bb,),
        in_specs=[
            pl.BlockSpec((bb, lp, d), lambda i: (i, 0, 0)),
            pl.BlockSpec((bb, l, l), lambda i: (i, 0, 0)),
            pl.BlockSpec((4 * _LP, d), lambda i: (0, 0)),
        ],
        out_specs=pl.BlockSpec((bb, l, d), lambda i: (i, 0, 0)),
        out_shape=jax.ShapeDtypeStruct((b, l, d), jnp.float32),
    )(h_p, adj, a_rep)


def kernel(inputs, adj, mask_item, item, embedding, a_0, a_1, a_2, a_3):
    b, l = inputs.shape
    _, dim = embedding.shape
    idx_pad = jnp.pad(inputs.astype(jnp.int32), ((0, 0), (0, _LP - l)))
    idx_flat = idx_pad.reshape(-1)

    gather = _make_sc_gather(b * _LP, dim)
    h_flat = gather(embedding, idx_flat)
    h_p = h_flat.reshape(b, _LP, dim)

    a4 = jnp.concatenate([a_0.T, a_1.T, a_2.T, a_3.T], axis=0)
    a_rep = jnp.repeat(a4, _LP, axis=0)
    return _attention_tc(h_p, adj, a_rep, bb=16)


# speedup vs baseline: 1.1852x; 1.1852x over previous
import functools

import jax
import jax.numpy as jnp
from jax import lax
from jax.experimental import pallas as pl
from jax.experimental.pallas import tpu as pltpu
from jax.experimental.pallas import tpu_sc as plsc

_ALPHA = 0.2
_NEG = -9e15
_LP = 64


def _make_sc_gather(n_rows, dim):
    info = plsc.get_sparse_core_info()
    nc, ns = info.num_cores, info.num_subcores
    nw = nc * ns
    assert n_rows % nw == 0
    b_per_w = n_rows // nw
    ch = 256
    assert b_per_w % ch == 0
    n_chunks = b_per_w // ch
    mesh = plsc.VectorSubcoreMesh(core_axis_name="c", subcore_axis_name="s")

    @functools.partial(
        pl.kernel,
        mesh=mesh,
        out_type=jax.ShapeDtypeStruct((n_rows, dim), jnp.float32),
        scratch_types=[
            pltpu.VMEM((b_per_w,), jnp.int32),
            pltpu.VMEM((2, ch, dim), jnp.float32),
            pltpu.SemaphoreType.DMA,
            pltpu.SemaphoreType.DMA,
        ],
    )
    def gather_kernel(table_hbm, idx_hbm, out_hbm, idx_v, rows_v, sem0, sem1):
        wid = lax.axis_index("s") * nc + lax.axis_index("c")
        base = wid * b_per_w
        sems = [sem0, sem1]
        pltpu.sync_copy(idx_hbm.at[pl.ds(base, b_per_w)], idx_v)
        copies = [None, None]
        copies[0] = pltpu.async_copy(
            table_hbm.at[idx_v.at[pl.ds(0, ch)]], rows_v.at[0], sems[0]
        )
        for c in range(n_chunks):
            nxt = c + 1
            if nxt < n_chunks:
                copies[nxt % 2] = pltpu.async_copy(
                    table_hbm.at[idx_v.at[pl.ds(nxt * ch, ch)]],
                    rows_v.at[nxt % 2],
                    sems[nxt % 2],
                )
            copies[c % 2].wait()
            pltpu.sync_copy(rows_v.at[c % 2], out_hbm.at[pl.ds(base + c * ch, ch)])

    return gather_kernel


def _make_attn_body(bb, l, d):
    def body(h_ref, adj_ref, a_ref, out_ref):
        a_rep = a_ref[...]
        hbs = []
        es = []
        for s in range(bb):
            hb = h_ref[s].astype(jnp.bfloat16)
            hbs.append(hb)
            ha = jnp.concatenate([hb, hb, hb, hb], axis=0) * a_rep
            es.append(
                lax.dot_general(
                    ha, hb, (((1,), (1,)), ((), ())),
                    preferred_element_type=jnp.float32,
                )
            )
        e = jnp.stack(es, axis=0)
        e = jnp.where(e >= 0, e, _ALPHA * e)
        adj_p = jnp.pad(adj_ref[...], ((0, 0), (0, _LP - l), (0, _LP - l)))
        att = jnp.full((bb, _LP, _LP), _NEG, dtype=jnp.float32)
        for k in range(4):
            att = jnp.where(adj_p == (k + 1), e[:, k * _LP : (k + 1) * _LP], att)
        m = jnp.max(att, axis=-1, keepdims=True)
        col_ok = lax.broadcasted_iota(jnp.int32, (1, _LP, _LP), 2) < l
        p = jnp.where(col_ok, jnp.exp(att - m), 0.0)
        attb = (p / jnp.sum(p, axis=-1, keepdims=True)).astype(jnp.bfloat16)
        for s in range(bb):
            out = lax.dot_general(
                attb[s], hbs[s], (((1,), (0,)), ((), ())),
                preferred_element_type=jnp.float32,
            )
            out_ref[s] = out[:l]

    return body


def _attention_tc(h_p, adj, a_rep, bb):
    b, lp, d = h_p.shape
    l = adj.shape[-1]
    return pl.pallas_call(
        _make_attn_body(bb, l, d),
        grid=(b // bb,),
        in_specs=[
            pl.BlockSpec((bb, lp, d), lambda i: (i, 0, 0)),
            pl.BlockSpec((bb, l, l), lambda i: (i, 0, 0)),
            pl.BlockSpec((4 * _LP, d), lambda i: (0, 0)),
        ],
        out_specs=pl.BlockSpec((bb, l, d), lambda i: (i, 0, 0)),
        out_shape=jax.ShapeDtypeStruct((b, l, d), jnp.float32),
    )(h_p, adj, a_rep)


def kernel(inputs, adj, mask_item, item, embedding, a_0, a_1, a_2, a_3):
    b, l = inputs.shape
    _, dim = embedding.shape
    idx_pad = jnp.pad(inputs.astype(jnp.int32), ((0, 0), (0, _LP - l)))
    idx_flat = idx_pad.reshape(-1)

    gather = _make_sc_gather(b * _LP, dim)
    h_flat = gather(embedding, idx_flat)
    h_p = h_flat.reshape(b, _LP, dim)

    a4 = jnp.concatenate([a_0.T, a_1.T, a_2.T, a_3.T], axis=0)
    a_rep = jnp.repeat(a4, _LP, axis=0).astype(jnp.bfloat16)
    return _attention_tc(h_p, adj, a_rep, bb=16)


# speedup vs baseline: 1.1900x; 1.0041x over previous
import functools

import jax
import jax.numpy as jnp
from jax import lax
from jax.experimental import pallas as pl
from jax.experimental.pallas import tpu as pltpu
from jax.experimental.pallas import tpu_sc as plsc

_ALPHA = 0.2
_NEG = -9e15
_LP = 64


def _make_sc_gather(n_rows, dim):
    info = plsc.get_sparse_core_info()
    nc, ns = info.num_cores, info.num_subcores
    nw = nc * ns
    assert n_rows % nw == 0
    b_per_w = n_rows // nw
    ch = 256
    assert b_per_w % ch == 0
    n_chunks = b_per_w // ch
    mesh = plsc.VectorSubcoreMesh(core_axis_name="c", subcore_axis_name="s")

    @functools.partial(
        pl.kernel,
        mesh=mesh,
        out_type=jax.ShapeDtypeStruct((n_rows, dim), jnp.float32),
        scratch_types=[
            pltpu.VMEM((b_per_w,), jnp.int32),
            pltpu.VMEM((2, ch, dim), jnp.float32),
            pltpu.SemaphoreType.DMA,
            pltpu.SemaphoreType.DMA,
        ],
    )
    def gather_kernel(table_hbm, idx_hbm, out_hbm, idx_v, rows_v, sem0, sem1):
        wid = lax.axis_index("s") * nc + lax.axis_index("c")
        base = wid * b_per_w
        sems = [sem0, sem1]
        pltpu.sync_copy(idx_hbm.at[pl.ds(base, b_per_w)], idx_v)
        copies = [None, None]
        copies[0] = pltpu.async_copy(
            table_hbm.at[idx_v.at[pl.ds(0, ch)]], rows_v.at[0], sems[0]
        )
        for c in range(n_chunks):
            nxt = c + 1
            if nxt < n_chunks:
                copies[nxt % 2] = pltpu.async_copy(
                    table_hbm.at[idx_v.at[pl.ds(nxt * ch, ch)]],
                    rows_v.at[nxt % 2],
                    sems[nxt % 2],
                )
            copies[c % 2].wait()
            pltpu.sync_copy(rows_v.at[c % 2], out_hbm.at[pl.ds(base + c * ch, ch)])

    return gather_kernel


def _make_attn_body(bb, l, d):
    def body(h_ref, adj_ref, a_ref, out_ref):
        a_rep = a_ref[...]
        adj_p = jnp.pad(adj_ref[...], ((0, 0), (0, _LP - l), (0, _LP - l)))
        hbs = []
        atts = []
        for s in range(bb):
            hb = h_ref[s].astype(jnp.bfloat16)
            hbs.append(hb)
            ha = jnp.concatenate([hb, hb, hb, hb], axis=0) * a_rep
            e = lax.dot_general(
                ha, hb, (((1,), (1,)), ((), ())),
                preferred_element_type=jnp.float32,
            )
            adj = adj_p[s]
            att_s = jnp.full((_LP, _LP), _NEG, dtype=jnp.float32)
            for k in range(4):
                att_s = jnp.where(adj == (k + 1), e[k * _LP : (k + 1) * _LP], att_s)
            atts.append(att_s)
        att = jnp.stack(atts, axis=0)
        att = jnp.where(att >= 0, att, _ALPHA * att)
        m = jnp.max(att, axis=-1, keepdims=True)
        col_ok = lax.broadcasted_iota(jnp.int32, (1, _LP, _LP), 2) < l
        p = jnp.where(col_ok, jnp.exp(att - m), 0.0)
        attb = (p / jnp.sum(p, axis=-1, keepdims=True)).astype(jnp.bfloat16)
        for s in range(bb):
            out = lax.dot_general(
                attb[s], hbs[s], (((1,), (0,)), ((), ())),
                preferred_element_type=jnp.float32,
            )
            out_ref[s] = out[:l]

    return body


def _attention_tc(h_p, adj, a_rep, bb):
    b, lp, d = h_p.shape
    l = adj.shape[-1]
    return pl.pallas_call(
        _make_attn_body(bb, l, d),
        grid=(b // bb,),
        in_specs=[
            pl.BlockSpec((bb, lp, d), lambda i: (i, 0, 0)),
            pl.BlockSpec((bb, l, l), lambda i: (i, 0, 0)),
            pl.BlockSpec((4 * _LP, d), lambda i: (0, 0)),
        ],
        out_specs=pl.BlockSpec((bb, l, d), lambda i: (i, 0, 0)),
        out_shape=jax.ShapeDtypeStruct((b, l, d), jnp.float32),
    )(h_p, adj, a_rep)


def kernel(inputs, adj, mask_item, item, embedding, a_0, a_1, a_2, a_3):
    b, l = inputs.shape
    _, dim = embedding.shape
    idx_pad = jnp.pad(inputs.astype(jnp.int32), ((0, 0), (0, _LP - l)))
    idx_flat = idx_pad.reshape(-1)

    gather = _make_sc_gather(b * _LP, dim)
    h_flat = gather(embedding, idx_flat)
    h_p = h_flat.reshape(b, _LP, dim)

    a4 = jnp.concatenate([a_0.T, a_1.T, a_2.T, a_3.T], axis=0)
    a_rep = jnp.repeat(a4, _LP, axis=0).astype(jnp.bfloat16)
    return _attention_tc(h_p, adj, a_rep, bb=16)


# speedup vs baseline: 5.1123x; 4.2960x over previous
import functools

import jax
import jax.numpy as jnp
from jax import lax
from jax.experimental import pallas as pl
from jax.experimental.pallas import tpu as pltpu
from jax.experimental.pallas import tpu_sc as plsc

_ALPHA = 0.2
_NEG = -9e15
_LP = 64


def _make_sc_gather(n_rows, dim):
    info = plsc.get_sparse_core_info()
    nc, ns = info.num_cores, info.num_subcores
    nw = nc * ns
    assert n_rows % nw == 0
    b_per_w = n_rows // nw
    ch = 400
    assert b_per_w % ch == 0
    n_chunks = b_per_w // ch
    mesh = plsc.VectorSubcoreMesh(core_axis_name="c", subcore_axis_name="s")

    @functools.partial(
        pl.kernel,
        mesh=mesh,
        out_type=jax.ShapeDtypeStruct((n_rows, dim), jnp.float32),
        scratch_types=[
            pltpu.VMEM((b_per_w,), jnp.int32),
            pltpu.VMEM((2, ch, dim), jnp.float32),
            pltpu.SemaphoreType.DMA,
            pltpu.SemaphoreType.DMA,
        ],
    )
    def gather_kernel(table_hbm, idx_hbm, out_hbm, idx_v, rows_v, sem0, sem1):
        wid = lax.axis_index("s") * nc + lax.axis_index("c")
        base = wid * b_per_w
        sems = [sem0, sem1]
        pltpu.sync_copy(idx_hbm.at[pl.ds(base, b_per_w)], idx_v)
        copies = [None, None]
        copies[0] = pltpu.async_copy(
            table_hbm.at[idx_v.at[pl.ds(0, ch)]], rows_v.at[0], sems[0]
        )
        for c in range(n_chunks):
            nxt = c + 1
            if nxt < n_chunks:
                copies[nxt % 2] = pltpu.async_copy(
                    table_hbm.at[idx_v.at[pl.ds(nxt * ch, ch)]],
                    rows_v.at[nxt % 2],
                    sems[nxt % 2],
                )
            copies[c % 2].wait()
            pltpu.sync_copy(rows_v.at[c % 2], out_hbm.at[pl.ds(base + c * ch, ch)])

    return gather_kernel


def _make_attn_body(bb, l, d):
    def body(h_ref, adj_ref, a_ref, out_ref):
        a_rep = a_ref[...]
        adj_p = jnp.pad(adj_ref[...], ((0, 0), (0, _LP - l), (0, _LP - l)))
        hb_all = jnp.pad(
            h_ref[...].astype(jnp.bfloat16), ((0, 0), (0, _LP - l), (0, 0))
        )
        hbs = []
        atts = []
        for s in range(bb):
            hb = hb_all[s]
            hbs.append(hb)
            ha = jnp.concatenate([hb, hb, hb, hb], axis=0) * a_rep
            e = lax.dot_general(
                ha, hb, (((1,), (1,)), ((), ())),
                preferred_element_type=jnp.float32,
            )
            adj = adj_p[s]
            att_s = jnp.full((_LP, _LP), _NEG, dtype=jnp.float32)
            for k in range(4):
                att_s = jnp.where(adj == (k + 1), e[k * _LP : (k + 1) * _LP], att_s)
            atts.append(att_s)
        att = jnp.stack(atts, axis=0)
        att = jnp.where(att >= 0, att, _ALPHA * att)
        m = jnp.max(att, axis=-1, keepdims=True)
        col_ok = lax.broadcasted_iota(jnp.int32, (1, _LP, _LP), 2) < l
        p = jnp.where(col_ok, jnp.exp(att - m), 0.0)
        attb = (p / jnp.sum(p, axis=-1, keepdims=True)).astype(jnp.bfloat16)
        for s in range(bb):
            out = lax.dot_general(
                attb[s], hbs[s], (((1,), (0,)), ((), ())),
                preferred_element_type=jnp.float32,
            )
            out_ref[s] = out[:l]

    return body


def _attention_tc(h, adj, a_rep, bb):
    b, l, d = h.shape
    return pl.pallas_call(
        _make_attn_body(bb, l, d),
        grid=(b // bb,),
        in_specs=[
            pl.BlockSpec((bb, l, d), lambda i: (i, 0, 0)),
            pl.BlockSpec((bb, l, l), lambda i: (i, 0, 0)),
            pl.BlockSpec((4 * _LP, d), lambda i: (0, 0)),
        ],
        out_specs=pl.BlockSpec((bb, l, d), lambda i: (i, 0, 0)),
        out_shape=jax.ShapeDtypeStruct((b, l, d), jnp.float32),
    )(h, adj, a_rep)


def kernel(inputs, adj, mask_item, item, embedding, a_0, a_1, a_2, a_3):
    b, l = inputs.shape
    _, dim = embedding.shape
    idx_flat = inputs.reshape(-1).astype(jnp.int32)

    gather = _make_sc_gather(b * l, dim)
    h_flat = gather(embedding, idx_flat)
    h = h_flat.reshape(b, l, dim)

    a4 = jnp.concatenate([a_0.T, a_1.T, a_2.T, a_3.T], axis=0)
    a_rep = jnp.repeat(a4, _LP, axis=0).astype(jnp.bfloat16)
    return _attention_tc(h, adj, a_rep, bb=16)


# speedup vs baseline: 5.8005x; 1.1346x over previous
import functools

import jax
import jax.numpy as jnp
from jax import lax
from jax.experimental import pallas as pl
from jax.experimental.pallas import tpu as pltpu
from jax.experimental.pallas import tpu_sc as plsc

_ALPHA = 0.2
_NEG = -9e15
_LP = 64


def _make_sc_gather(n_rows, dim):
    info = plsc.get_sparse_core_info()
    nc, ns = info.num_cores, info.num_subcores
    nw = nc * ns
    assert n_rows % nw == 0
    b_per_w = n_rows // nw
    ch = 400
    assert b_per_w % ch == 0
    n_chunks = b_per_w // ch
    mesh = plsc.VectorSubcoreMesh(core_axis_name="c", subcore_axis_name="s")

    @functools.partial(
        pl.kernel,
        mesh=mesh,
        out_type=jax.ShapeDtypeStruct((n_rows, dim), jnp.float32),
        scratch_types=[
            pltpu.VMEM((b_per_w,), jnp.int32),
            pltpu.VMEM((2, ch, dim), jnp.float32),
            pltpu.SemaphoreType.DMA,
            pltpu.SemaphoreType.DMA,
        ],
    )
    def gather_kernel(table_hbm, idx_hbm, out_hbm, idx_v, rows_v, sem0, sem1):
        wid = lax.axis_index("s") * nc + lax.axis_index("c")
        base = wid * b_per_w
        sems = [sem0, sem1]
        pltpu.sync_copy(idx_hbm.at[pl.ds(base, b_per_w)], idx_v)
        copies = [None, None]
        copies[0] = pltpu.async_copy(
            table_hbm.at[idx_v.at[pl.ds(0, ch)]], rows_v.at[0], sems[0]
        )
        for c in range(n_chunks):
            nxt = c + 1
            if nxt < n_chunks:
                copies[nxt % 2] = pltpu.async_copy(
                    table_hbm.at[idx_v.at[pl.ds(nxt * ch, ch)]],
                    rows_v.at[nxt % 2],
                    sems[nxt % 2],
                )
            copies[c % 2].wait()
            pltpu.sync_copy(rows_v.at[c % 2], out_hbm.at[pl.ds(base + c * ch, ch)])

    return gather_kernel


def _make_attn_body(bb, l, d):
    def body(h_ref, adj_ref, a_ref, out_ref):
        a_rep = a_ref[...]
        adj_p = jnp.pad(adj_ref[...], ((0, 0), (0, _LP - l), (0, _LP - l)))
        hb_all = jnp.pad(
            h_ref[...].astype(jnp.bfloat16), ((0, 0), (0, _LP - l), (0, 0))
        )
        hbs = []
        atts = []
        for s in range(bb):
            hb = hb_all[s]
            hbs.append(hb)
            ha = jnp.concatenate([hb, hb, hb, hb], axis=0) * a_rep
            e = lax.dot_general(
                ha, hb, (((1,), (1,)), ((), ())),
                preferred_element_type=jnp.float32,
            )
            adj = adj_p[s]
            att_s = jnp.full((_LP, _LP), _NEG, dtype=jnp.float32)
            for k in range(4):
                att_s = jnp.where(adj == (k + 1), e[k * _LP : (k + 1) * _LP], att_s)
            atts.append(att_s)
        att = jnp.stack(atts, axis=0)
        att = jnp.where(att >= 0, att, _ALPHA * att)
        m = jnp.max(att, axis=-1, keepdims=True)
        col_ok = lax.broadcasted_iota(jnp.int32, (1, _LP, _LP), 2) < l
        p = jnp.where(col_ok, jnp.exp(att - m), 0.0)
        attb = (p / jnp.sum(p, axis=-1, keepdims=True)).astype(jnp.bfloat16)
        for s in range(bb):
            out = lax.dot_general(
                attb[s], hbs[s], (((1,), (0,)), ((), ())),
                preferred_element_type=jnp.float32,
            )
            out_ref[s] = out[:l]

    return body


def _attention_tc(h, adj, a_rep, bb):
    b, l, d = h.shape
    return pl.pallas_call(
        _make_attn_body(bb, l, d),
        grid=(b // bb,),
        in_specs=[
            pl.BlockSpec((bb, l, d), lambda i: (i, 0, 0)),
            pl.BlockSpec((bb, l, l), lambda i: (i, 0, 0)),
            pl.BlockSpec((4 * _LP, d), lambda i: (0, 0)),
        ],
        out_specs=pl.BlockSpec((bb, l, d), lambda i: (i, 0, 0)),
        out_shape=jax.ShapeDtypeStruct((b, l, d), jnp.float32),
    )(h, adj, a_rep)


def kernel(inputs, adj, mask_item, item, embedding, a_0, a_1, a_2, a_3):
    b, l = inputs.shape
    _, dim = embedding.shape
    idx_flat = inputs.reshape(-1).astype(jnp.int32)

    gather = _make_sc_gather(b * l, dim)
    h_flat = gather(embedding, idx_flat)
    h = h_flat.reshape(b, l, dim)

    a4 = jnp.concatenate([a_0.T, a_1.T, a_2.T, a_3.T], axis=0)
    a_rep = jnp.repeat(a4, _LP, axis=0).astype(jnp.bfloat16)
    return _attention_tc(h, adj, a_rep, bb=32)


# speedup vs baseline: 6.2392x; 1.0756x over previous
import functools

import jax
import jax.numpy as jnp
from jax import lax
from jax.experimental import pallas as pl
from jax.experimental.pallas import tpu as pltpu
from jax.experimental.pallas import tpu_sc as plsc

_ALPHA = 0.2
_NEG = -9e15
_LP = 64


def _make_sc_gather(n_rows, dim):
    info = plsc.get_sparse_core_info()
    nc, ns = info.num_cores, info.num_subcores
    nw = nc * ns
    assert n_rows % nw == 0
    b_per_w = n_rows // nw
    ch = 400
    assert b_per_w % ch == 0
    n_chunks = b_per_w // ch
    mesh = plsc.VectorSubcoreMesh(core_axis_name="c", subcore_axis_name="s")

    @functools.partial(
        pl.kernel,
        mesh=mesh,
        out_type=jax.ShapeDtypeStruct((n_rows, dim), jnp.float32),
        scratch_types=[
            pltpu.VMEM((b_per_w,), jnp.int32),
            pltpu.VMEM((2, ch, dim), jnp.float32),
            pltpu.SemaphoreType.DMA,
            pltpu.SemaphoreType.DMA,
        ],
    )
    def gather_kernel(table_hbm, idx_hbm, out_hbm, idx_v, rows_v, sem0, sem1):
        wid = lax.axis_index("s") * nc + lax.axis_index("c")
        base = wid * b_per_w
        sems = [sem0, sem1]
        pltpu.sync_copy(idx_hbm.at[pl.ds(base, b_per_w)], idx_v)
        copies = [None, None]
        copies[0] = pltpu.async_copy(
            table_hbm.at[idx_v.at[pl.ds(0, ch)]], rows_v.at[0], sems[0]
        )
        for c in range(n_chunks):
            nxt = c + 1
            if nxt < n_chunks:
                copies[nxt % 2] = pltpu.async_copy(
                    table_hbm.at[idx_v.at[pl.ds(nxt * ch, ch)]],
                    rows_v.at[nxt % 2],
                    sems[nxt % 2],
                )
            copies[c % 2].wait()
            pltpu.sync_copy(rows_v.at[c % 2], out_hbm.at[pl.ds(base + c * ch, ch)])

    return gather_kernel


def _make_attn_body(bb, l, d):
    def body(h_ref, adj_ref, a_ref, out_ref):
        a_rep = a_ref[...]
        adj_p = jnp.pad(adj_ref[...], ((0, 0), (0, _LP - l), (0, _LP - l)))
        hb_all = jnp.pad(
            h_ref[...].astype(jnp.bfloat16), ((0, 0), (0, _LP - l), (0, 0))
        )
        hbs = []
        atts = []
        for s in range(bb):
            hb = hb_all[s]
            hbs.append(hb)
            ha = jnp.concatenate([hb, hb, hb, hb], axis=0) * a_rep
            e = lax.dot_general(
                ha, hb, (((1,), (1,)), ((), ())),
                preferred_element_type=jnp.float32,
            )
            adj = adj_p[s]
            att_s = jnp.full((_LP, _LP), _NEG, dtype=jnp.float32)
            for k in range(4):
                att_s = jnp.where(adj == (k + 1), e[k * _LP : (k + 1) * _LP], att_s)
            atts.append(att_s)
        att = jnp.stack(atts, axis=0)
        att = jnp.where(att >= 0, att, _ALPHA * att)
        m = jnp.max(att, axis=-1, keepdims=True)
        col_ok = lax.broadcasted_iota(jnp.int32, (1, _LP, _LP), 2) < l
        p = jnp.where(col_ok, jnp.exp(att - m), 0.0)
        attb = (p / jnp.sum(p, axis=-1, keepdims=True)).astype(jnp.bfloat16)
        for s in range(bb):
            out = lax.dot_general(
                attb[s], hbs[s], (((1,), (0,)), ((), ())),
                preferred_element_type=jnp.float32,
            )
            out_ref[s] = out[:l]

    return body


def _attention_tc(h, adj, a_rep, bb):
    b, l, d = h.shape
    return pl.pallas_call(
        _make_attn_body(bb, l, d),
        grid=(b // bb,),
        in_specs=[
            pl.BlockSpec((bb, l, d), lambda i: (i, 0, 0)),
            pl.BlockSpec((bb, l, l), lambda i: (i, 0, 0)),
            pl.BlockSpec((4 * _LP, d), lambda i: (0, 0)),
        ],
        out_specs=pl.BlockSpec((bb, l, d), lambda i: (i, 0, 0)),
        out_shape=jax.ShapeDtypeStruct((b, l, d), jnp.float32),
    )(h, adj, a_rep)


def kernel(inputs, adj, mask_item, item, embedding, a_0, a_1, a_2, a_3):
    b, l = inputs.shape
    _, dim = embedding.shape
    idx_flat = inputs.reshape(-1).astype(jnp.int32)

    gather = _make_sc_gather(b * l, dim)
    h_flat = gather(embedding, idx_flat)
    h = h_flat.reshape(b, l, dim)

    a4 = jnp.concatenate([a_0.T, a_1.T, a_2.T, a_3.T], axis=0)
    a_rep = jnp.repeat(a4, _LP, axis=0).astype(jnp.bfloat16)
    return _attention_tc(h, adj, a_rep, bb=64)


# speedup vs baseline: 7.2774x; 1.1664x over previous
import functools

import jax
import jax.numpy as jnp
from jax import lax
from jax.experimental import pallas as pl
from jax.experimental.pallas import tpu as pltpu
from jax.experimental.pallas import tpu_sc as plsc

_ALPHA = 0.2
_NEG = -9e15
_LP = 64


def _make_sc_gather(n_rows, dim):
    info = plsc.get_sparse_core_info()
    nc, ns = info.num_cores, info.num_subcores
    nw = nc * ns
    assert n_rows % nw == 0
    b_per_w = n_rows // nw
    ch = 256
    assert b_per_w % ch == 0
    n_chunks = b_per_w // ch
    mesh = plsc.VectorSubcoreMesh(core_axis_name="c", subcore_axis_name="s")

    @functools.partial(
        pl.kernel,
        mesh=mesh,
        out_type=jax.ShapeDtypeStruct((n_rows, dim), jnp.float32),
        scratch_types=[
            pltpu.VMEM((b_per_w,), jnp.int32),
            pltpu.VMEM((2, ch, dim), jnp.float32),
            pltpu.SemaphoreType.DMA,
            pltpu.SemaphoreType.DMA,
        ],
    )
    def gather_kernel(table_hbm, idx_hbm, out_hbm, idx_v, rows_v, sem0, sem1):
        wid = lax.axis_index("s") * nc + lax.axis_index("c")
        base = wid * b_per_w
        sems = [sem0, sem1]
        pltpu.sync_copy(idx_hbm.at[pl.ds(base, b_per_w)], idx_v)
        copies = [None, None]
        copies[0] = pltpu.async_copy(
            table_hbm.at[idx_v.at[pl.ds(0, ch)]], rows_v.at[0], sems[0]
        )
        for c in range(n_chunks):
            nxt = c + 1
            if nxt < n_chunks:
                copies[nxt % 2] = pltpu.async_copy(
                    table_hbm.at[idx_v.at[pl.ds(nxt * ch, ch)]],
                    rows_v.at[nxt % 2],
                    sems[nxt % 2],
                )
            copies[c % 2].wait()
            pltpu.sync_copy(rows_v.at[c % 2], out_hbm.at[pl.ds(base + c * ch, ch)])

    return gather_kernel


def _make_attn_body(bb, l, d):
    def body(h_ref, adj_ref, a_ref, out_ref):
        a_rep = a_ref[...]
        adj_p = jnp.pad(adj_ref[...], ((0, 0), (0, _LP - l), (0, _LP - l)))
        hb_all = h_ref[...].astype(jnp.bfloat16)
        hbs = []
        atts = []
        for s in range(bb):
            hb = hb_all[s]
            hbs.append(hb)
            ha = jnp.concatenate([hb, hb, hb, hb], axis=0) * a_rep
            e = lax.dot_general(
                ha, hb, (((1,), (1,)), ((), ())),
                preferred_element_type=jnp.float32,
            )
            adj = adj_p[s]
            att_s = jnp.full((_LP, _LP), _NEG, dtype=jnp.float32)
            for k in range(4):
                att_s = jnp.where(adj == (k + 1), e[k * _LP : (k + 1) * _LP], att_s)
            atts.append(att_s)
        att = jnp.stack(atts, axis=0)
        att = jnp.where(att >= 0, att, _ALPHA * att)
        m = jnp.max(att, axis=-1, keepdims=True)
        col_ok = lax.broadcasted_iota(jnp.int32, (1, _LP, _LP), 2) < l
        p = jnp.where(col_ok, jnp.exp(att - m), 0.0)
        attb = (p / jnp.sum(p, axis=-1, keepdims=True)).astype(jnp.bfloat16)
        for s in range(bb):
            out = lax.dot_general(
                attb[s], hbs[s], (((1,), (0,)), ((), ())),
                preferred_element_type=jnp.float32,
            )
            out_ref[s] = out[:l]

    return body


def _attention_tc(h, adj, a_rep, bb):
    b, lp, d = h.shape
    l = adj.shape[-1]
    return pl.pallas_call(
        _make_attn_body(bb, l, d),
        grid=(b // bb,),
        in_specs=[
            pl.BlockSpec((bb, lp, d), lambda i: (i, 0, 0)),
            pl.BlockSpec((bb, l, l), lambda i: (i, 0, 0)),
            pl.BlockSpec((4 * _LP, d), lambda i: (0, 0)),
        ],
        out_specs=pl.BlockSpec((bb, l, d), lambda i: (i, 0, 0)),
        out_shape=jax.ShapeDtypeStruct((b, l, d), jnp.float32),
    )(h, adj, a_rep)


def kernel(inputs, adj, mask_item, item, embedding, a_0, a_1, a_2, a_3):
    b, l = inputs.shape
    _, dim = embedding.shape
    idx32 = inputs.astype(jnp.int32)
    idx_pad = jnp.concatenate([idx32, idx32[:, : _LP - l]], axis=1)
    idx_flat = idx_pad.reshape(-1)

    gather = _make_sc_gather(b * _LP, dim)
    h_flat = gather(embedding, idx_flat)
    h = h_flat.reshape(b, _LP, dim)

    a4 = jnp.concatenate([a_0.T, a_1.T, a_2.T, a_3.T], axis=0)
    a_rep = jnp.repeat(a4, _LP, axis=0).astype(jnp.bfloat16)
    return _attention_tc(h, adj, a_rep, bb=64)
